# Initial kernel scaffold; baseline (speedup 1.0000x reference)
import jax
import jax.numpy as jnp
from jax.experimental import pallas as pl
from jax.experimental.pallas import tpu as pltpu

_K = 34992
_TK = 2048
_KSPLIT = 9
_LAST_TILE = 17
_LAST_VALID = _K - _LAST_TILE * _TK


def _l1_kernel(x_ref, w_ref, o_ref):
    j = pl.program_id(0)
    k = pl.program_id(1)

    @pl.when(k == 0)
    def _():
        o_ref[...] = jnp.zeros_like(o_ref)

    xb = x_ref[...]
    t = j * _KSPLIT + k
    valid = jnp.where(t == _LAST_TILE, _LAST_VALID, _TK)
    col = jax.lax.broadcasted_iota(jnp.int32, xb.shape, 1)
    xb = jnp.where(col < valid, xb, 0.0)

    o_ref[0] += jnp.dot(
        xb.astype(jnp.bfloat16),
        w_ref[...].astype(jnp.bfloat16),
        preferred_element_type=jnp.float32,
    )


def _tail_kernel(p_ref, b1_ref, w2_ref, b2_ref, w3_ref, b3_ref,
                 w4_ref, b4_ref, w5_ref, b5_ref, o_ref):
    h = p_ref[0] + p_ref[1] + b1_ref[...]
    h = jnp.maximum(h, 0.0)
    for w_ref, b_ref, relu in ((w2_ref, b2_ref, True),
                               (w3_ref, b3_ref, True),
                               (w4_ref, b4_ref, True),
                               (w5_ref, b5_ref, False)):
        h = jnp.dot(h, w_ref[...], preferred_element_type=jnp.float32)
        h = h + b_ref[...]
        if relu:
            h = jnp.maximum(h, 0.0)
    o_ref[...] = h


def kernel(x, w1, b1, w2, b2, w3, b3, w4, b4, w5, b5):
    B = x.shape[0]
    xf = x.reshape(B, -1)
    N1 = w1.shape[1]

    partials = pl.pallas_call(
        _l1_kernel,
        out_shape=jax.ShapeDtypeStruct((2, B, N1), jnp.float32),
        grid=(2, _KSPLIT),
        in_specs=[
            pl.BlockSpec((B, _TK), lambda j, k: (0, j * _KSPLIT + k)),
            pl.BlockSpec((_TK, N1), lambda j, k: (j * _KSPLIT + k, 0)),
        ],
        out_specs=pl.BlockSpec((1, B, N1), lambda j, k: (j, 0, 0)),
        compiler_params=pltpu.CompilerParams(
            dimension_semantics=("parallel", "arbitrary"),
            vmem_limit_bytes=100 << 20,
        ),
    )(xf, w1)

    MB = B // 2
    out = pl.pallas_call(
        _tail_kernel,
        out_shape=jax.ShapeDtypeStruct((B, w5.shape[1]), jnp.float32),
        grid=(2,),
        in_specs=[
            pl.BlockSpec((2, MB, N1), lambda i: (0, i, 0)),
            pl.BlockSpec(b1.shape, lambda i: (0, 0)),
            pl.BlockSpec(w2.shape, lambda i: (0, 0)),
            pl.BlockSpec(b2.shape, lambda i: (0, 0)),
            pl.BlockSpec(w3.shape, lambda i: (0, 0)),
            pl.BlockSpec(b3.shape, lambda i: (0, 0)),
            pl.BlockSpec(w4.shape, lambda i: (0, 0)),
            pl.BlockSpec(b4.shape, lambda i: (0, 0)),
            pl.BlockSpec(w5.shape, lambda i: (0, 0)),
            pl.BlockSpec(b5.shape, lambda i: (0, 0)),
        ],
        out_specs=pl.BlockSpec((MB, w5.shape[1]), lambda i: (i, 0)),
        compiler_params=pltpu.CompilerParams(
            dimension_semantics=("parallel",),
        ),
    )(partials, b1, w2, b2, w3, b3, w4, b4, w5, b5)
    return out


# speedup vs baseline: 1.2397x; 1.2397x over previous
import jax
import jax.numpy as jnp
from jax.experimental import pallas as pl
from jax.experimental.pallas import tpu as pltpu

_K = 34992
_TK = 1024
_KSPLIT = 18
_LAST_X_TILE = _K // _TK


def _l1_kernel(x_ref, w_ref, o_ref):
    j = pl.program_id(0)
    k = pl.program_id(1)

    @pl.when(k == 0)
    def _():
        o_ref[...] = jnp.zeros_like(o_ref)

    xb = x_ref[...]
    t = j * _KSPLIT + k
    valid = jnp.clip(_K - t * _TK, 0, _TK)
    col = jax.lax.broadcasted_iota(jnp.int32, xb.shape, 1)
    xb = jnp.where(col < valid, xb, 0.0)

    o_ref[0] += jnp.dot(
        xb.astype(jnp.bfloat16),
        w_ref[...].astype(jnp.bfloat16),
        preferred_element_type=jnp.float32,
    )


def _tail_kernel(p_ref, b1_ref, w2_ref, b2_ref, w3_ref, b3_ref,
                 w4_ref, b4_ref, w5_ref, b5_ref, o_ref):
    h = p_ref[0] + p_ref[1] + b1_ref[...]
    h = jnp.maximum(h, 0.0)
    for w_ref, b_ref, relu in ((w2_ref, b2_ref, True),
                               (w3_ref, b3_ref, True),
                               (w4_ref, b4_ref, True),
                               (w5_ref, b5_ref, False)):
        h = jnp.dot(h, w_ref[...], preferred_element_type=jnp.float32)
        h = h + b_ref[...]
        if relu:
            h = jnp.maximum(h, 0.0)
    o_ref[...] = h


def kernel(x, w1, b1, w2, b2, w3, b3, w4, b4, w5, b5):
    B = x.shape[0]
    xf = x.reshape(B, -1)
    N1 = w1.shape[1]

    partials = pl.pallas_call(
        _l1_kernel,
        out_shape=jax.ShapeDtypeStruct((2, B, N1), jnp.float32),
        grid=(2, _KSPLIT),
        in_specs=[
            pl.BlockSpec(
                (B, _TK),
                lambda j, k: (0, jnp.minimum(j * _KSPLIT + k, _LAST_X_TILE))),
            pl.BlockSpec((_TK, N1), lambda j, k: (j * _KSPLIT + k, 0)),
        ],
        out_specs=pl.BlockSpec((1, B, N1), lambda j, k: (j, 0, 0)),
        compiler_params=pltpu.CompilerParams(
            dimension_semantics=("parallel", "arbitrary"),
            vmem_limit_bytes=100 << 20,
        ),
    )(xf, w1)

    MB = B // 2
    out = pl.pallas_call(
        _tail_kernel,
        out_shape=jax.ShapeDtypeStruct((B, w5.shape[1]), jnp.float32),
        grid=(2,),
        in_specs=[
            pl.BlockSpec((2, MB, N1), lambda i: (0, i, 0)),
            pl.BlockSpec(b1.shape, lambda i: (0, 0)),
            pl.BlockSpec(w2.shape, lambda i: (0, 0)),
            pl.BlockSpec(b2.shape, lambda i: (0, 0)),
            pl.BlockSpec(w3.shape, lambda i: (0, 0)),
            pl.BlockSpec(b3.shape, lambda i: (0, 0)),
            pl.BlockSpec(w4.shape, lambda i: (0, 0)),
            pl.BlockSpec(b4.shape, lambda i: (0, 0)),
            pl.BlockSpec(w5.shape, lambda i: (0, 0)),
            pl.BlockSpec(b5.shape, lambda i: (0, 0)),
        ],
        out_specs=pl.BlockSpec((MB, w5.shape[1]), lambda i: (i, 0)),
        compiler_params=pltpu.CompilerParams(
            dimension_semantics=("parallel",),
        ),
    )(partials, b1, w2, b2, w3, b3, w4, b4, w5, b5)
    return out


# speedup vs baseline: 1.3678x; 1.1033x over previous
import jax
import jax.numpy as jnp
from jax.experimental import pallas as pl
from jax.experimental.pallas import tpu as pltpu

_HS = 12
_TK = _HS * 108
_NCHUNK = 27
_KSPLIT = 14
_B = 1024


def _slab(t):
    t = jnp.minimum(t, _NCHUNK - 1)
    return t // 9, (t % 9) * _HS


def _l1_kernel(x_hbm, w_ref, o_ref, xbuf, sem):
    j = pl.program_id(0)
    k = pl.program_id(1)

    def x_copy(t, slot):
        c, h0 = _slab(t)
        return pltpu.make_async_copy(
            x_hbm.at[:, c, pl.ds(h0, _HS), :], xbuf.at[slot], sem.at[slot])

    @pl.when(k == 0)
    def _():
        o_ref[...] = jnp.zeros_like(o_ref)
        x_copy(j * _KSPLIT, 0).start()

    @pl.when(k + 1 < _KSPLIT)
    def _():
        x_copy(j * _KSPLIT + k + 1, (k + 1) % 2).start()

    slot = k % 2
    x_copy(j * _KSPLIT + k, slot).wait()
    xb = xbuf[slot].astype(jnp.bfloat16)
    xr = xb.reshape(_B, _TK)
    o_ref[0] += jnp.dot(
        xr, w_ref[...].astype(jnp.bfloat16),
        preferred_element_type=jnp.float32,
    )


def _tail_kernel(p_ref, b1_ref, w2_ref, b2_ref, w3_ref, b3_ref,
                 w4_ref, b4_ref, w5_ref, b5_ref, o_ref):
    h = p_ref[0] + p_ref[1] + b1_ref[...]
    h = jnp.maximum(h, 0.0)
    for w_ref, b_ref, relu in ((w2_ref, b2_ref, True),
                               (w3_ref, b3_ref, True),
                               (w4_ref, b4_ref, True),
                               (w5_ref, b5_ref, False)):
        h = jnp.dot(h, w_ref[...], preferred_element_type=jnp.float32)
        h = h + b_ref[...]
        if relu:
            h = jnp.maximum(h, 0.0)
    o_ref[...] = h


def kernel(x, w1, b1, w2, b2, w3, b3, w4, b4, w5, b5):
    B = x.shape[0]
    N1 = w1.shape[1]

    partials = pl.pallas_call(
        _l1_kernel,
        out_shape=jax.ShapeDtypeStruct((2, B, N1), jnp.float32),
        grid=(2, _KSPLIT),
        in_specs=[
            pl.BlockSpec(memory_space=pl.ANY),
            pl.BlockSpec((_TK, N1), lambda j, k: (j * _KSPLIT + k, 0)),
        ],
        out_specs=pl.BlockSpec((1, B, N1), lambda j, k: (j, 0, 0)),
        scratch_shapes=[
            pltpu.VMEM((2, B, _HS, 108), jnp.float32),
            pltpu.SemaphoreType.DMA((2,)),
        ],
        compiler_params=pltpu.CompilerParams(
            dimension_semantics=("parallel", "arbitrary"),
            vmem_limit_bytes=62 << 20,
        ),
    )(x, w1)

    MB = B // 2
    out = pl.pallas_call(
        _tail_kernel,
        out_shape=jax.ShapeDtypeStruct((B, w5.shape[1]), jnp.float32),
        grid=(2,),
        in_specs=[
            pl.BlockSpec((2, MB, N1), lambda i: (0, i, 0)),
            pl.BlockSpec(b1.shape, lambda i: (0, 0)),
            pl.BlockSpec(w2.shape, lambda i: (0, 0)),
            pl.BlockSpec(b2.shape, lambda i: (0, 0)),
            pl.BlockSpec(w3.shape, lambda i: (0, 0)),
            pl.BlockSpec(b3.shape, lambda i: (0, 0)),
            pl.BlockSpec(w4.shape, lambda i: (0, 0)),
            pl.BlockSpec(b4.shape, lambda i: (0, 0)),
            pl.BlockSpec(w5.shape, lambda i: (0, 0)),
            pl.BlockSpec(b5.shape, lambda i: (0, 0)),
        ],
        out_specs=pl.BlockSpec((MB, w5.shape[1]), lambda i: (i, 0)),
        compiler_params=pltpu.CompilerParams(
            dimension_semantics=("parallel",),
        ),
    )(partials, b1, w2, b2, w3, b3, w4, b4, w5, b5)
    return out


# speedup vs baseline: 1.4497x; 1.0599x over previous
import jax
import jax.numpy as jnp
from jax.experimental import pallas as pl
from jax.experimental.pallas import tpu as pltpu

_HS = 12
_TK = _HS * 108
_NCHUNK = 27
_B = 1024


def _slab(t):
    t = jnp.minimum(t, _NCHUNK - 1)
    return t // 9, (t % 9) * _HS


def _net_kernel(x_hbm, w1_ref, b1_ref, w2_ref, b2_ref, w3_ref, b3_ref,
                w4_ref, b4_ref, w5_ref, b5_ref, o_ref, xbuf, sem, acc):
    k = pl.program_id(0)

    def x_copy(t, slot):
        c, h0 = _slab(t)
        return pltpu.make_async_copy(
            x_hbm.at[:, c, pl.ds(h0, _HS), :], xbuf.at[slot], sem.at[slot])

    @pl.when(k == 0)
    def _():
        acc[...] = jnp.zeros_like(acc)
        x_copy(0, 0).start()

    @pl.when(k + 1 < _NCHUNK)
    def _():
        x_copy(k + 1, (k + 1) % 2).start()

    slot = k % 2
    x_copy(k, slot).wait()
    xb = xbuf[slot].astype(jnp.bfloat16)
    xr = xb.reshape(_B, _TK)
    acc[...] += jnp.dot(
        xr, w1_ref[...].astype(jnp.bfloat16),
        preferred_element_type=jnp.float32,
    )

    @pl.when(k == _NCHUNK - 1)
    def _():
        h = jnp.maximum(acc[...] + b1_ref[...], 0.0)
        for w_ref, b_ref, relu in ((w2_ref, b2_ref, True),
                                   (w3_ref, b3_ref, True),
                                   (w4_ref, b4_ref, True),
                                   (w5_ref, b5_ref, False)):
            h = jnp.dot(h.astype(jnp.bfloat16), w_ref[...].astype(jnp.bfloat16),
                        preferred_element_type=jnp.float32)
            h = h + b_ref[...]
            if relu:
                h = jnp.maximum(h, 0.0)
        o_ref[...] = h


def kernel(x, w1, b1, w2, b2, w3, b3, w4, b4, w5, b5):
    B = x.shape[0]
    N1 = w1.shape[1]

    def _const(k):
        return (0, 0)

    return pl.pallas_call(
        _net_kernel,
        out_shape=jax.ShapeDtypeStruct((B, w5.shape[1]), jnp.float32),
        grid=(_NCHUNK,),
        in_specs=[
            pl.BlockSpec(memory_space=pl.ANY),
            pl.BlockSpec((_TK, N1), lambda k: (k, 0)),
            pl.BlockSpec(b1.shape, _const),
            pl.BlockSpec(w2.shape, _const),
            pl.BlockSpec(b2.shape, _const),
            pl.BlockSpec(w3.shape, _const),
            pl.BlockSpec(b3.shape, _const),
            pl.BlockSpec(w4.shape, _const),
            pl.BlockSpec(b4.shape, _const),
            pl.BlockSpec(w5.shape, _const),
            pl.BlockSpec(b5.shape, _const),
        ],
        out_specs=pl.BlockSpec((B, w5.shape[1]), _const),
        scratch_shapes=[
            pltpu.VMEM((2, B, _HS, 108), jnp.float32),
            pltpu.SemaphoreType.DMA((2,)),
            pltpu.VMEM((B, N1), jnp.float32),
        ],
        compiler_params=pltpu.CompilerParams(
            dimension_semantics=("arbitrary",),
            vmem_limit_bytes=62 << 20,
        ),
    )(x, w1, b1, w2, b2, w3, b3, w4, b4, w5, b5)
